# Initial kernel scaffold; baseline (speedup 1.0000x reference)
import jax
import jax.numpy as jnp
from jax.experimental import pallas as pl


def kernel(node_attrs, positions, shifts, edge_attr, edge_index, batch_ids, W_node, b_node, W_edge, b_edge, Wv, bv, Wm, bm, W1, b1, W2, b2, W3, b3, W4, b4):
    raise NotImplementedError("write your pallas kernel here")


# speedup vs baseline: 1.5255x; 1.5255x over previous
import functools

import jax
import jax.numpy as jnp
import numpy as np
from jax import lax
from jax.experimental import pallas as pl
from jax.experimental.pallas import tpu as pltpu
from jax.experimental.pallas import tpu_sc as plsc

_HID = 32
_HALF = 16
_NEG = -1e30
_INDS = np.array([[0, 1, 2, 3, 4, 5], [1, 6, 7, 8, 9, 10], [2, 7, 11, 12, 13, 14],
                  [3, 8, 12, 15, 16, 17], [4, 9, 13, 16, 18, 19], [5, 10, 14, 17, 19, 20]])

@functools.cache
def _mesh():
    return plsc.VectorSubcoreMesh(core_axis_name="c", subcore_axis_name="s")


def _posgather_kernel(pos_hbm, s_hbm, r_hbm, ps_hbm, pr_hbm,
                      idxs, idxr, bufs, bufr, sem1, sem2):
    c = lax.axis_index("c")
    sid = lax.axis_index("s")
    wid = sid * 2 + c
    e_pad = s_hbm.shape[0]
    per_w = e_pad // 32
    nchunks = per_w // 128

    def body(i, carry):
        base = wid * per_w + i * 128
        pltpu.sync_copy(s_hbm.at[pl.ds(base, 128)], idxs.at[0])
        pltpu.sync_copy(r_hbm.at[pl.ds(base, 128)], idxr.at[0])
        cp1 = pltpu.async_copy(pos_hbm.at[idxs.at[0]], bufs, sem1)
        cp2 = pltpu.async_copy(pos_hbm.at[idxr.at[0]], bufr, sem2)
        cp1.wait()
        cp2.wait()
        pltpu.sync_copy(bufs, ps_hbm.at[pl.ds(base, 128)])
        pltpu.sync_copy(bufr, pr_hbm.at[pl.ds(base, 128)])
        return carry

    lax.fori_loop(0, nchunks, body, 0)


def _posgather(pos_pad, s_pad, r_pad):
    e_pad = s_pad.shape[0]
    out = jax.ShapeDtypeStruct((e_pad, 16), jnp.float32)
    k = pl.kernel(
        _posgather_kernel,
        out_type=(out, out),
        mesh=_mesh(),
        compiler_params=pltpu.CompilerParams(use_tc_tiling_on_sc=False, needs_layout_passes=False),
        scratch_types=[
            pltpu.VMEM((1, 128), jnp.int32),
            pltpu.VMEM((1, 128), jnp.int32),
            pltpu.VMEM((128, 16), jnp.float32),
            pltpu.VMEM((128, 16), jnp.float32),
            pltpu.SemaphoreType.DMA,
            pltpu.SemaphoreType.DMA,
        ],
    )
    return k(pos_pad, s_pad, r_pad)


def _edgeproj_kernel(n_real, blk, ps_ref, pr_ref, ea_ref, w_ref, b_ref, *out_refs):
    i = pl.program_id(0)
    d = pr_ref[...] - ps_ref[...]
    ss = jnp.sum(d * d, axis=1, keepdims=True)
    ln = jnp.sqrt(ss)
    inv = 1.0 / (ln + 1e-9)
    u = d * inv
    lane = lax.broadcasted_iota(jnp.int32, d.shape, 1)
    x = jnp.where(lane < 3, u,
                  jnp.where(lane == 3, ln,
                            jnp.where(lane == 4, ea_ref[...], 0.0)))
    p = lax.dot_general(x, w_ref[...], (((1,), (0,)), ((), ())),
                        preferred_element_type=jnp.float32) + b_ref[...]
    row = i * blk + lax.broadcasted_iota(jnp.int32, (d.shape[0], 1), 0)
    valid = row < n_real
    for q, oref in enumerate(out_refs):
        oref[...] = jnp.where(valid, p[:, 16 * q:16 * q + 16], _NEG)


def _edgeproj(ps, pr, ea_pad, w_pad, b_pad, n_real):
    e_pad = ps.shape[0]
    blk = 2048
    grid = e_pad // blk
    out = [jax.ShapeDtypeStruct((e_pad, 16), jnp.float32) for _ in range(12)]
    return pl.pallas_call(
        functools.partial(_edgeproj_kernel, n_real, blk),
        grid=(grid,),
        in_specs=[
            pl.BlockSpec((blk, 16), lambda i: (i, 0)),
            pl.BlockSpec((blk, 16), lambda i: (i, 0)),
            pl.BlockSpec((blk, 1), lambda i: (i, 0)),
            pl.BlockSpec((16, 192), lambda i: (0, 0)),
            pl.BlockSpec((1, 192), lambda i: (0, 0)),
        ],
        out_specs=[pl.BlockSpec((blk, 16), lambda i: (i, 0)) for _ in range(12)],
        out_shape=out,
    )(ps, pr, ea_pad, w_pad, b_pad)


def _embed_kernel(na_ref, w_ref, b_ref, s0, s1, r0, r1):
    t = na_ref[...] * w_ref[...] + b_ref[...]
    s0[...] = t[:, 0:32]
    s1[...] = t[:, 32:64]
    r0[...] = t[:, 64:96]
    r1[...] = t[:, 96:128]


def _embed_tables(node_attrs, w_comb, b_comb):
    n = node_attrs.shape[0]
    bn = 1000
    out = [jax.ShapeDtypeStruct((n, 32), jnp.float32) for _ in range(4)]
    return pl.pallas_call(
        _embed_kernel,
        grid=(n // bn,),
        in_specs=[
            pl.BlockSpec((bn, 1), lambda i: (i, 0)),
            pl.BlockSpec((1, 128), lambda i: (0, 0)),
            pl.BlockSpec((1, 128), lambda i: (0, 0)),
        ],
        out_specs=[pl.BlockSpec((bn, 32), lambda i: (i, 0)) for _ in range(4)],
        out_shape=out,
    )(node_attrs, w_comb, b_comb)


def _nodeproj_kernel(xp_ref, y0_ref, y1_ref, w_ref, xo, s0, s1, r0, r1):
    x = xp_ref[...] + jnp.concatenate([y0_ref[...], y1_ref[...]], axis=1)
    xo[...] = x
    t = lax.dot_general(x, w_ref[...], (((1,), (0,)), ((), ())),
                        preferred_element_type=jnp.float32)
    s0[...] = t[:, 0:32]
    s1[...] = t[:, 32:64]
    r0[...] = t[:, 64:96]
    r1[...] = t[:, 96:128]


def _node_tables(xprev, y0, y1, w_all):
    n = xprev.shape[0]
    bn = 1000
    out = [jax.ShapeDtypeStruct((n, 32), jnp.float32) for _ in range(5)]
    return pl.pallas_call(
        _nodeproj_kernel,
        grid=(n // bn,),
        in_specs=[
            pl.BlockSpec((bn, 32), lambda i: (i, 0)),
            pl.BlockSpec((bn, 16), lambda i: (i, 0)),
            pl.BlockSpec((bn, 16), lambda i: (i, 0)),
            pl.BlockSpec((32, 128), lambda i: (0, 0)),
        ],
        out_specs=[pl.BlockSpec((bn, 32), lambda i: (i, 0))] +
                  [pl.BlockSpec((bn, 32), lambda i: (i, 0)) for _ in range(4)],
        out_shape=out,
    )(xprev, y0, y1, w_all)


def _softplus_sc(gv):
    t = jnp.exp(gv)
    y = 1.0 + t
    bi = plsc.bitcast(y, jnp.int32)
    ex = (bi >> 23) - 127
    mb = plsc.bitcast((bi & 0x007FFFFF) | 0x3F800000, jnp.float32)
    big = mb > 1.4142135
    m2 = jnp.where(big, mb * 0.5, mb)
    ef = ex.astype(jnp.float32) + jnp.where(big, 1.0, 0.0)
    sf = (m2 - 1.0) / (m2 + 1.0)
    z = sf * sf
    lm = sf * (2.0 + z * (0.6666666667 + z * (0.4 + z * 0.2857142857)))
    ly = ef * 0.69314718056 + lm
    return jnp.where(gv > 15.0, gv, ly)


def _mp_kernel(s_hbm, r_hbm, sx0, sx1, rx0, rx1, pv0, pv1, pm0, pm1,
               y0_hbm, y1_hbm,
               idxs, idxr, bufS, bufR, bufEv, bufEm, bufMsg, zbuf, acc,
               semS, semR, semE, semF):
    c = lax.axis_index("c")
    sid = lax.axis_index("s")
    e_pad = s_hbm.shape[0]
    n = acc.shape[0]
    per_tile = e_pad // 16
    nchunks = per_tile // 128
    rows_per_tile = n // 16
    zchunk = rows_per_tile // 10

    def run(s_ref, r_ref, pv_ref, pm_ref, y_ref):
        def zfill(i, carry):
            zbuf[i, :] = jnp.zeros((16,), jnp.float32)
            return carry
        lax.fori_loop(0, zchunk, zfill, 0)

        def zcopy(j, carry):
            pltpu.sync_copy(zbuf, acc.at[pl.ds(sid * rows_per_tile + j * zchunk, zchunk)])
            return carry
        lax.fori_loop(0, 10, zcopy, 0)
        plsc.subcore_barrier()

        def chunk(i, carry):
            base = sid * per_tile + i * 128
            pltpu.sync_copy(s_hbm.at[pl.ds(base, 128)], idxs.at[0])
            pltpu.sync_copy(r_hbm.at[pl.ds(base, 128)], idxr.at[0])
            cpS = pltpu.async_copy(s_ref.at[idxs.at[0]], bufS, semS)
            cpR = pltpu.async_copy(r_ref.at[idxr.at[0]], bufR, semR)
            cpE = pltpu.async_copy(pv_ref.at[pl.ds(base, 128)], bufEv, semE)
            cpF = pltpu.async_copy(pm_ref.at[pl.ds(base, 128)], bufEm, semF)
            cpS.wait()
            cpR.wait()
            cpE.wait()
            cpF.wait()

            def edge(e, carry2):
                av = bufS[e, 0:16]
                am = bufS[e, 16:32]
                bv = bufR[e, 0:16]
                bm = bufR[e, 16:32]
                gv = av + bv + bufEv[e, :]
                gm = am + bm + bufEm[e, :]
                sg = 1.0 / (1.0 + jnp.exp(-gm))
                bufMsg[e, :] = _softplus_sc(gv) * sg
                return carry2

            lax.fori_loop(0, 128, edge, 0)
            pltpu.sync_copy(bufMsg, acc.at[idxr.at[0]], add=True)
            return carry

        lax.fori_loop(0, nchunks, chunk, 0)
        plsc.subcore_barrier()
        pltpu.sync_copy(acc.at[pl.ds(sid * rows_per_tile, rows_per_tile)],
                        y_ref.at[pl.ds(sid * rows_per_tile, rows_per_tile)])

    @pl.when(c == 0)
    def _():
        run(sx0, rx0, pv0, pm0, y0_hbm)

    @pl.when(c == 1)
    def _():
        run(sx1, rx1, pv1, pm1, y1_hbm)


def _mp_layer(s_pad, r_pad, sx0, sx1, rx0, rx1, pv0, pv1, pm0, pm1, n):
    out = jax.ShapeDtypeStruct((n, 16), jnp.float32)
    k = pl.kernel(
        _mp_kernel,
        out_type=(out, out),
        mesh=_mesh(),
        compiler_params=pltpu.CompilerParams(use_tc_tiling_on_sc=False, needs_layout_passes=False),
        scratch_types=[
            pltpu.VMEM((1, 128), jnp.int32),
            pltpu.VMEM((1, 128), jnp.int32),
            pltpu.VMEM((128, 32), jnp.float32),
            pltpu.VMEM((128, 32), jnp.float32),
            pltpu.VMEM((128, 16), jnp.float32),
            pltpu.VMEM((128, 16), jnp.float32),
            pltpu.VMEM((128, 16), jnp.float32),
            pltpu.VMEM((625, 16), jnp.float32),
            pltpu.VMEM_SHARED((n, 16), jnp.float32),
            pltpu.SemaphoreType.DMA,
            pltpu.SemaphoreType.DMA,
            pltpu.SemaphoreType.DMA,
            pltpu.SemaphoreType.DMA,
        ],
    )
    return k(s_pad, r_pad, sx0, sx1, rx0, rx1, pv0, pv1, pm0, pm1)


def _head_kernel(nblocks, xp_ref, y0_ref, y1_ref, b_ref,
                 w1, b1, w2, b2, w3, b3, w4, b4, out_ref, sums, cnts):
    i = pl.program_id(0)

    @pl.when(i == 0)
    def _():
        sums[...] = jnp.zeros_like(sums)
        cnts[...] = jnp.zeros_like(cnts)

    x = xp_ref[...] + jnp.concatenate([y0_ref[...], y1_ref[...]], axis=1)
    g = lax.broadcasted_iota(jnp.int32, (x.shape[0], 256), 1)
    oh = (g == b_ref[...]).astype(jnp.float32)
    sums[...] += lax.dot_general(oh, x, (((0,), (0,)), ((), ())),
                                 preferred_element_type=jnp.float32)
    cnts[...] += lax.dot_general(oh, jnp.ones_like(x), (((0,), (0,)), ((), ())),
                                 preferred_element_type=jnp.float32)

    @pl.when(i == nblocks - 1)
    def _():
        gf = sums[...] / jnp.maximum(cnts[...], 1.0)

        def sp(v):
            return jnp.maximum(v, 0.0) + jnp.log1p(jnp.exp(-jnp.abs(v)))

        h = sp(jnp.dot(gf, w1[...], preferred_element_type=jnp.float32) + b1[...])
        h = sp(jnp.dot(h, w2[...], preferred_element_type=jnp.float32) + b2[...])
        h = sp(jnp.dot(h, w3[...], preferred_element_type=jnp.float32) + b3[...])
        out_ref[...] = jnp.dot(h, w4[...], preferred_element_type=jnp.float32) + b4[...]


def _head(xprev, y0, y1, batch2d, w1, b1, w2, b2, w3, b3, w4, b4):
    n = xprev.shape[0]
    bn = 1000
    nblocks = n // bn
    return pl.pallas_call(
        functools.partial(_head_kernel, nblocks),
        grid=(nblocks,),
        in_specs=[
            pl.BlockSpec((bn, 32), lambda i: (i, 0)),
            pl.BlockSpec((bn, 16), lambda i: (i, 0)),
            pl.BlockSpec((bn, 16), lambda i: (i, 0)),
            pl.BlockSpec((bn, 1), lambda i: (i, 0)),
            pl.BlockSpec((32, 128), lambda i: (0, 0)),
            pl.BlockSpec((1, 128), lambda i: (0, 0)),
            pl.BlockSpec((128, 64), lambda i: (0, 0)),
            pl.BlockSpec((1, 64), lambda i: (0, 0)),
            pl.BlockSpec((64, 32), lambda i: (0, 0)),
            pl.BlockSpec((1, 32), lambda i: (0, 0)),
            pl.BlockSpec((32, 21), lambda i: (0, 0)),
            pl.BlockSpec((1, 21), lambda i: (0, 0)),
        ],
        out_specs=pl.BlockSpec((256, 21), lambda i: (0, 0)),
        out_shape=jax.ShapeDtypeStruct((256, 21), jnp.float32),
        scratch_shapes=[
            pltpu.VMEM((256, 32), jnp.float32),
            pltpu.VMEM((256, 32), jnp.float32),
        ],
    )(xprev, y0, y1, batch2d, w1, b1, w2, b2, w3, b3, w4, b4)


def kernel(node_attrs, positions, shifts, edge_attr, edge_index, batch_ids,
           W_node, b_node, W_edge, b_edge, Wv, bv, Wm, bm,
           W1, b1, W2, b2, W3, b3, W4, b4):
    n = node_attrs.shape[0]
    e = edge_index.shape[1]
    e_pad = ((e + 4095) // 4096) * 4096
    mp = Wv.shape[0]

    s = edge_index[0].astype(jnp.int32)
    r = edge_index[1].astype(jnp.int32)
    pad = e_pad - e
    pad_idx = (jnp.arange(pad, dtype=jnp.int32) % n)
    s_pad = jnp.concatenate([s, pad_idx])
    r_pad = jnp.concatenate([r, pad_idx])
    pos_pad = jnp.pad(positions, ((0, 0), (0, 13)))
    ea_pad = jnp.pad(edge_attr, ((0, pad), (0, 0)))

    wp_cols = []
    bp_cols = []
    for l in range(mp):
        for wq, bq in ((Wv[l], bv[l]), (Wm[l], bm[l])):
            we = W_edge @ wq[2 * _HID:3 * _HID]
            be = b_edge @ wq[2 * _HID:3 * _HID] + bq
            for h in range(2):
                wp_cols.append(we[:, 16 * h:16 * h + 16])
                bp_cols.append(be[16 * h:16 * h + 16])
    w_pe = jnp.concatenate(wp_cols, axis=1)
    w_pad = jnp.zeros((16, 192), jnp.float32).at[0:5, :].set(w_pe)
    b_pad = jnp.concatenate(bp_cols).reshape(1, 192)

    def table_w(l):
        ws_v, wr_v = Wv[l][0:_HID], Wv[l][_HID:2 * _HID]
        ws_m, wr_m = Wm[l][0:_HID], Wm[l][_HID:2 * _HID]
        cols = []
        for a, bcol in ((ws_v, ws_m), (wr_v, wr_m)):
            for h in range(2):
                cols.append(a[:, 16 * h:16 * h + 16])
                cols.append(bcol[:, 16 * h:16 * h + 16])
        return jnp.concatenate(cols, axis=1)

    w_tab = [table_w(l) for l in range(mp)]
    w_comb = (W_node @ w_tab[0]).reshape(1, 128)
    b_comb = (b_node @ w_tab[0]).reshape(1, 128)

    ps, pr = _posgather(pos_pad, s_pad, r_pad)
    slabs = _edgeproj(ps, pr, ea_pad, w_pad, b_pad, e)

    sx0, sx1, rx0, rx1 = _embed_tables(node_attrs, w_comb, b_comb)
    y0, y1 = _mp_layer(s_pad, r_pad, sx0, sx1, rx0, rx1,
                       slabs[0], slabs[1], slabs[2], slabs[3], n)
    xprev = jnp.zeros((n, _HID), jnp.float32)
    for l in range(1, mp):
        xprev, sx0, sx1, rx0, rx1 = _node_tables(xprev, y0, y1, w_tab[l])
        y0, y1 = _mp_layer(s_pad, r_pad, sx0, sx1, rx0, rx1,
                           slabs[4 * l], slabs[4 * l + 1],
                           slabs[4 * l + 2], slabs[4 * l + 3], n)

    out = _head(xprev, y0, y1, batch_ids.astype(jnp.int32).reshape(n, 1),
                W1, b1.reshape(1, -1), W2, b2.reshape(1, -1),
                W3, b3.reshape(1, -1), W4, b4.reshape(1, -1))
    return out[:, _INDS]


# speedup vs baseline: 1.8784x; 1.2314x over previous
import functools

import jax
import jax.numpy as jnp
import numpy as np
from jax import lax
from jax.experimental import pallas as pl
from jax.experimental.pallas import tpu as pltpu
from jax.experimental.pallas import tpu_sc as plsc

_HID = 32
_HALF = 16
_NT = 100352
_BN = 1024
_INDS = np.array([[0, 1, 2, 3, 4, 5], [1, 6, 7, 8, 9, 10], [2, 7, 11, 12, 13, 14],
                  [3, 8, 12, 15, 16, 17], [4, 9, 13, 16, 18, 19], [5, 10, 14, 17, 19, 20]])

_SC_PARAMS = dict(
    compiler_params=pltpu.CompilerParams(use_tc_tiling_on_sc=False,
                                         needs_layout_passes=False),
)


@functools.cache
def _mesh():
    return plsc.VectorSubcoreMesh(core_axis_name="c", subcore_axis_name="s")


def _posgather_kernel(pos_hbm, s_hbm, r_hbm, ps_hbm, pr_hbm,
                      idxs, idxr, bufs, bufr, sem1, sem2):
    c = lax.axis_index("c")
    sid = lax.axis_index("s")
    wid = sid * 2 + c
    e_pad = s_hbm.shape[0]
    per_w = e_pad // 32
    nchunks = per_w // 128

    def issue(i, b):
        base = wid * per_w + i * 256 + b * 128
        pltpu.sync_copy(s_hbm.at[pl.ds(base, 128)], idxs.at[b])
        pltpu.sync_copy(r_hbm.at[pl.ds(base, 128)], idxr.at[b])
        sem = sem1 if b == 0 else sem2
        pltpu.async_copy(pos_hbm.at[idxs.at[b]], bufs.at[b], sem)
        pltpu.async_copy(pos_hbm.at[idxr.at[b]], bufr.at[b], sem)

    def drain(b):
        sem = sem1 if b == 0 else sem2
        pltpu.make_async_copy(pos_hbm.at[idxs.at[b]], bufs.at[b], sem).wait()
        pltpu.make_async_copy(pos_hbm.at[idxr.at[b]], bufr.at[b], sem).wait()

    issue(0, 0)
    issue(0, 1)

    def body(j, carry):
        for b in range(2):
            base = wid * per_w + j * 256 + b * 128
            drain(b)
            pltpu.sync_copy(bufs.at[b], ps_hbm.at[pl.ds(base, 128)])
            pltpu.sync_copy(bufr.at[b], pr_hbm.at[pl.ds(base, 128)])

            @pl.when(j + 1 < nchunks // 2)
            def _():
                issue(j + 1, b)
        return carry

    lax.fori_loop(0, nchunks // 2, body, 0)


def _posgather(pos_pad, s_pad, r_pad):
    e_pad = s_pad.shape[0]
    out = jax.ShapeDtypeStruct((e_pad, 16), jnp.float32)
    k = pl.kernel(
        _posgather_kernel,
        out_type=(out, out),
        mesh=_mesh(),
        scratch_types=[
            pltpu.VMEM((2, 128), jnp.int32),
            pltpu.VMEM((2, 128), jnp.int32),
            pltpu.VMEM((2, 128, 16), jnp.float32),
            pltpu.VMEM((2, 128, 16), jnp.float32),
            pltpu.SemaphoreType.DMA,
            pltpu.SemaphoreType.DMA,
        ],
        **_SC_PARAMS,
    )
    return k(pos_pad, s_pad, r_pad)


def _edgeproj_kernel(ps_ref, pr_ref, ea_ref, w_ref, b_ref, *out_refs):
    d = pr_ref[...] - ps_ref[...]
    ss = jnp.sum(d * d, axis=1, keepdims=True)
    ln = jnp.sqrt(ss)
    inv = 1.0 / (ln + 1e-9)
    u = d * inv
    lane = lax.broadcasted_iota(jnp.int32, d.shape, 1)
    x = jnp.where(lane < 3, u,
                  jnp.where(lane == 3, ln,
                            jnp.where(lane == 4, ea_ref[...], 0.0)))
    p = lax.dot_general(x, w_ref[...], (((1,), (0,)), ((), ())),
                        preferred_element_type=jnp.float32) + b_ref[...]
    for q, oref in enumerate(out_refs):
        oref[...] = p[:, 32 * q:32 * q + 32]


def _edgeproj(ps, pr, ea_pad, w_pad, b_pad):
    e_pad = ps.shape[0]
    blk = 4096
    grid = e_pad // blk
    out = [jax.ShapeDtypeStruct((e_pad, 32), jnp.float32) for _ in range(6)]
    return pl.pallas_call(
        _edgeproj_kernel,
        grid=(grid,),
        in_specs=[
            pl.BlockSpec((blk, 16), lambda i: (i, 0)),
            pl.BlockSpec((blk, 16), lambda i: (i, 0)),
            pl.BlockSpec((blk, 1), lambda i: (i, 0)),
            pl.BlockSpec((16, 192), lambda i: (0, 0)),
            pl.BlockSpec((1, 192), lambda i: (0, 0)),
        ],
        out_specs=[pl.BlockSpec((blk, 32), lambda i: (i, 0)) for _ in range(6)],
        out_shape=out,
    )(ps, pr, ea_pad, w_pad, b_pad)


def _embed_kernel(na_ref, w_ref, b_ref, s0, s1, r0, r1):
    t = na_ref[...] * w_ref[...] + b_ref[...]
    s0[...] = t[:, 0:32]
    s1[...] = t[:, 32:64]
    r0[...] = t[:, 64:96]
    r1[...] = t[:, 96:128]


def _embed_tables(node_attrs, w_comb, b_comb):
    n = node_attrs.shape[0]
    out = [jax.ShapeDtypeStruct((n, 32), jnp.float32) for _ in range(4)]
    return pl.pallas_call(
        _embed_kernel,
        grid=(n // _BN,),
        in_specs=[
            pl.BlockSpec((_BN, 1), lambda i: (i, 0)),
            pl.BlockSpec((1, 128), lambda i: (0, 0)),
            pl.BlockSpec((1, 128), lambda i: (0, 0)),
        ],
        out_specs=[pl.BlockSpec((_BN, 32), lambda i: (i, 0)) for _ in range(4)],
        out_shape=out,
    )(node_attrs, w_comb, b_comb)


def _nodeproj_kernel(xp_ref, y0_ref, y1_ref, w_ref, xo, s0, s1, r0, r1):
    x = xp_ref[...] + jnp.concatenate([y0_ref[...], y1_ref[...]], axis=1)
    xo[...] = x
    t = lax.dot_general(x, w_ref[...], (((1,), (0,)), ((), ())),
                        preferred_element_type=jnp.float32)
    s0[...] = t[:, 0:32]
    s1[...] = t[:, 32:64]
    r0[...] = t[:, 64:96]
    r1[...] = t[:, 96:128]


def _node_tables(xprev, y0, y1, w_all):
    n = xprev.shape[0]
    out = [jax.ShapeDtypeStruct((n, 32), jnp.float32) for _ in range(5)]
    return pl.pallas_call(
        _nodeproj_kernel,
        grid=(n // _BN,),
        in_specs=[
            pl.BlockSpec((_BN, 32), lambda i: (i, 0)),
            pl.BlockSpec((_BN, 16), lambda i: (i, 0)),
            pl.BlockSpec((_BN, 16), lambda i: (i, 0)),
            pl.BlockSpec((32, 128), lambda i: (0, 0)),
        ],
        out_specs=[pl.BlockSpec((_BN, 32), lambda i: (i, 0))] +
                  [pl.BlockSpec((_BN, 32), lambda i: (i, 0)) for _ in range(4)],
        out_shape=out,
    )(xprev, y0, y1, w_all)


def _softplus_sc(gv):
    t = jnp.exp(gv)
    y = 1.0 + t
    bi = plsc.bitcast(y, jnp.int32)
    ex = (bi >> 23) - 127
    mb = plsc.bitcast((bi & 0x007FFFFF) | 0x3F800000, jnp.float32)
    big = mb > 1.4142135
    m2 = jnp.where(big, mb * 0.5, mb)
    ef = ex.astype(jnp.float32) + jnp.where(big, 1.0, 0.0)
    sf = (m2 - 1.0) / (m2 + 1.0)
    z = sf * sf
    lm = sf * (2.0 + z * (0.6666666667 + z * (0.4 + z * 0.2857142857)))
    ly = ef * 0.69314718056 + lm
    return jnp.where(gv > 15.0, gv, ly)


def _mp_kernel(s_hbm, r_hbm, sx0, sx1, rx0, rx1, e0_hbm, e1_hbm,
               y0_hbm, y1_hbm,
               idxs, idxr, bufS, bufR, bufE, bufMsg, zbuf, acc,
               sem0, sem1):
    c = lax.axis_index("c")
    sid = lax.axis_index("s")
    e_pad = s_hbm.shape[0]
    n_acc = acc.shape[0]
    per_tile = e_pad // 16
    nchunks = per_tile // 128
    rows_per_tile = n_acc // 16
    zchunk = rows_per_tile // 64
    sems = (sem0, sem1)

    def run(s_ref, r_ref, e_ref, y_ref):
        def zfill(i, carry):
            zbuf[i, :] = jnp.zeros((16,), jnp.float32)
            return carry
        lax.fori_loop(0, zchunk, zfill, 0)

        def zcopy(j, carry):
            pltpu.sync_copy(zbuf, acc.at[pl.ds(sid * rows_per_tile + j * zchunk, zchunk)])
            return carry
        lax.fori_loop(0, 64, zcopy, 0)
        plsc.subcore_barrier()

        def issue(i, b):
            base = sid * per_tile + i * 128
            pltpu.sync_copy(s_hbm.at[pl.ds(base, 128)], idxs.at[b])
            pltpu.sync_copy(r_hbm.at[pl.ds(base, 128)], idxr.at[b])
            pltpu.async_copy(s_ref.at[idxs.at[b]], bufS.at[b], sems[b])
            pltpu.async_copy(r_ref.at[idxr.at[b]], bufR.at[b], sems[b])
            pltpu.async_copy(e_ref.at[pl.ds(base, 128)], bufE.at[b], sems[b])

        def drain(b):
            pltpu.make_async_copy(s_ref.at[idxs.at[b]], bufS.at[b], sems[b]).wait()
            pltpu.make_async_copy(r_ref.at[idxr.at[b]], bufR.at[b], sems[b]).wait()
            pltpu.make_async_copy(e_ref.at[pl.ds(0, 128)], bufE.at[b], sems[b]).wait()

        issue(0, 0)
        issue(1, 1)

        def body(j, carry):
            for b in range(2):
                drain(b)

                def edge(e, carry2):
                    av = bufS[b, e, 0:16]
                    am = bufS[b, e, 16:32]
                    bv = bufR[b, e, 0:16]
                    bm = bufR[b, e, 16:32]
                    gv = av + bv + bufE[b, e, 0:16]
                    gm = am + bm + bufE[b, e, 16:32]
                    sg = 1.0 / (1.0 + jnp.exp(-gm))
                    bufMsg[e, :] = _softplus_sc(gv) * sg
                    return carry2

                lax.fori_loop(0, 128, edge, 0)
                pltpu.sync_copy(bufMsg, acc.at[idxr.at[b]], add=True)

                @pl.when(2 * j + 2 + b < nchunks)
                def _():
                    issue(2 * j + 2 + b, b)
            return carry

        lax.fori_loop(0, nchunks // 2, body, 0)
        plsc.subcore_barrier()
        pltpu.sync_copy(acc.at[pl.ds(sid * rows_per_tile, rows_per_tile)],
                        y_ref.at[pl.ds(sid * rows_per_tile, rows_per_tile)])

    @pl.when(c == 0)
    def _():
        run(sx0, rx0, e0_hbm, y0_hbm)

    @pl.when(c == 1)
    def _():
        run(sx1, rx1, e1_hbm, y1_hbm)


def _mp_layer(s_pad, r_pad, sx0, sx1, rx0, rx1, e0, e1, n_t):
    out = jax.ShapeDtypeStruct((n_t, 16), jnp.float32)
    k = pl.kernel(
        _mp_kernel,
        out_type=(out, out),
        mesh=_mesh(),
        scratch_types=[
            pltpu.VMEM((2, 128), jnp.int32),
            pltpu.VMEM((2, 128), jnp.int32),
            pltpu.VMEM((2, 128, 32), jnp.float32),
            pltpu.VMEM((2, 128, 32), jnp.float32),
            pltpu.VMEM((2, 128, 32), jnp.float32),
            pltpu.VMEM((128, 16), jnp.float32),
            pltpu.VMEM((n_t // 1024, 16), jnp.float32),
            pltpu.VMEM_SHARED((n_t, 16), jnp.float32),
            pltpu.SemaphoreType.DMA,
            pltpu.SemaphoreType.DMA,
        ],
        **_SC_PARAMS,
    )
    return k(s_pad, r_pad, sx0, sx1, rx0, rx1, e0, e1)


def _head_kernel(nblocks, xp_ref, y0_ref, y1_ref, b_ref,
                 w1, b1, w2, b2, w3, b3, w4, b4, out_ref, sums, cnts):
    i = pl.program_id(0)

    @pl.when(i == 0)
    def _():
        sums[...] = jnp.zeros_like(sums)
        cnts[...] = jnp.zeros_like(cnts)

    x = xp_ref[...] + jnp.concatenate([y0_ref[...], y1_ref[...]], axis=1)
    g = lax.broadcasted_iota(jnp.int32, (x.shape[0], 256), 1)
    oh = (g == b_ref[...]).astype(jnp.float32)
    sums[...] += lax.dot_general(oh, x, (((0,), (0,)), ((), ())),
                                 preferred_element_type=jnp.float32)
    cnts[...] += lax.dot_general(oh, jnp.ones_like(x), (((0,), (0,)), ((), ())),
                                 preferred_element_type=jnp.float32)

    @pl.when(i == nblocks - 1)
    def _():
        gf = sums[...] / jnp.maximum(cnts[...], 1.0)

        def sp(v):
            return jnp.maximum(v, 0.0) + jnp.log1p(jnp.exp(-jnp.abs(v)))

        h = sp(jnp.dot(gf, w1[...], preferred_element_type=jnp.float32) + b1[...])
        h = sp(jnp.dot(h, w2[...], preferred_element_type=jnp.float32) + b2[...])
        h = sp(jnp.dot(h, w3[...], preferred_element_type=jnp.float32) + b3[...])
        out_ref[...] = jnp.dot(h, w4[...], preferred_element_type=jnp.float32) + b4[...]


def _head(xprev, y0, y1, batch2d, w1, b1, w2, b2, w3, b3, w4, b4):
    n = batch2d.shape[0]
    bn = 1000
    nblocks = n // bn
    return pl.pallas_call(
        functools.partial(_head_kernel, nblocks),
        grid=(nblocks,),
        in_specs=[
            pl.BlockSpec((bn, 32), lambda i: (i, 0)),
            pl.BlockSpec((bn, 16), lambda i: (i, 0)),
            pl.BlockSpec((bn, 16), lambda i: (i, 0)),
            pl.BlockSpec((bn, 1), lambda i: (i, 0)),
            pl.BlockSpec((32, 128), lambda i: (0, 0)),
            pl.BlockSpec((1, 128), lambda i: (0, 0)),
            pl.BlockSpec((128, 64), lambda i: (0, 0)),
            pl.BlockSpec((1, 64), lambda i: (0, 0)),
            pl.BlockSpec((64, 32), lambda i: (0, 0)),
            pl.BlockSpec((1, 32), lambda i: (0, 0)),
            pl.BlockSpec((32, 21), lambda i: (0, 0)),
            pl.BlockSpec((1, 21), lambda i: (0, 0)),
        ],
        out_specs=pl.BlockSpec((256, 21), lambda i: (0, 0)),
        out_shape=jax.ShapeDtypeStruct((256, 21), jnp.float32),
        scratch_shapes=[
            pltpu.VMEM((256, 32), jnp.float32),
            pltpu.VMEM((256, 32), jnp.float32),
        ],
    )(xprev, y0, y1, batch2d, w1, b1, w2, b2, w3, b3, w4, b4)


def kernel(node_attrs, positions, shifts, edge_attr, edge_index, batch_ids,
           W_node, b_node, W_edge, b_edge, Wv, bv, Wm, bm,
           W1, b1, W2, b2, W3, b3, W4, b4):
    n = node_attrs.shape[0]
    e = edge_index.shape[1]
    e_pad = ((e + 4095) // 4096) * 4096
    mp = Wv.shape[0]
    n_t = _NT

    s = edge_index[0].astype(jnp.int32)
    r = edge_index[1].astype(jnp.int32)
    pad = e_pad - e
    pad_idx = jnp.full((pad,), n, jnp.int32)
    s_pad = jnp.concatenate([s, pad_idx])
    r_pad = jnp.concatenate([r, pad_idx])
    pos_pad = jnp.pad(positions, ((0, n_t - n), (0, 13)))
    na_pad = jnp.pad(node_attrs, ((0, n_t - n), (0, 0)))
    ea_pad = jnp.pad(edge_attr, ((0, pad), (0, 0)))

    wp_cols = []
    bp_cols = []
    for l in range(mp):
        we_v = W_edge @ Wv[l][2 * _HID:3 * _HID]
        be_v = b_edge @ Wv[l][2 * _HID:3 * _HID] + bv[l]
        we_m = W_edge @ Wm[l][2 * _HID:3 * _HID]
        be_m = b_edge @ Wm[l][2 * _HID:3 * _HID] + bm[l]
        for h in range(2):
            wp_cols.append(we_v[:, 16 * h:16 * h + 16])
            wp_cols.append(we_m[:, 16 * h:16 * h + 16])
            bp_cols.append(be_v[16 * h:16 * h + 16])
            bp_cols.append(be_m[16 * h:16 * h + 16])
    w_pe = jnp.concatenate(wp_cols, axis=1)
    w_pad = jnp.zeros((16, 192), jnp.float32).at[0:5, :].set(w_pe)
    b_pad = jnp.concatenate(bp_cols).reshape(1, 192)

    def table_w(l):
        ws_v, wr_v = Wv[l][0:_HID], Wv[l][_HID:2 * _HID]
        ws_m, wr_m = Wm[l][0:_HID], Wm[l][_HID:2 * _HID]
        cols = []
        for a, bcol in ((ws_v, ws_m), (wr_v, wr_m)):
            for h in range(2):
                cols.append(a[:, 16 * h:16 * h + 16])
                cols.append(bcol[:, 16 * h:16 * h + 16])
        return jnp.concatenate(cols, axis=1)

    w_tab = [table_w(l) for l in range(mp)]
    w_comb = (W_node @ w_tab[0]).reshape(1, 128)
    b_comb = (b_node @ w_tab[0]).reshape(1, 128)

    ps, pr = _posgather(pos_pad, s_pad, r_pad)
    slabs = _edgeproj(ps, pr, ea_pad, w_pad, b_pad)

    sx0, sx1, rx0, rx1 = _embed_tables(na_pad, w_comb, b_comb)
    y0, y1 = _mp_layer(s_pad, r_pad, sx0, sx1, rx0, rx1,
                       slabs[0], slabs[1], n_t)
    xprev = jnp.zeros((n_t, _HID), jnp.float32)
    for l in range(1, mp):
        xprev, sx0, sx1, rx0, rx1 = _node_tables(xprev, y0, y1, w_tab[l])
        y0, y1 = _mp_layer(s_pad, r_pad, sx0, sx1, rx0, rx1,
                           slabs[2 * l], slabs[2 * l + 1], n_t)

    out = _head(xprev, y0, y1, batch_ids.astype(jnp.int32).reshape(n, 1),
                W1, b1.reshape(1, -1), W2, b2.reshape(1, -1),
                W3, b3.reshape(1, -1), W4, b4.reshape(1, -1))
    return out[:, _INDS]


# speedup vs baseline: 3.0965x; 1.6485x over previous
import functools

import jax
import jax.numpy as jnp
import numpy as np
from jax import lax
from jax.experimental import pallas as pl
from jax.experimental.pallas import tpu as pltpu
from jax.experimental.pallas import tpu_sc as plsc

_HID = 32
_HALF = 16
_NT = 100352
_BN = 1024
_INDS = np.array([[0, 1, 2, 3, 4, 5], [1, 6, 7, 8, 9, 10], [2, 7, 11, 12, 13, 14],
                  [3, 8, 12, 15, 16, 17], [4, 9, 13, 16, 18, 19], [5, 10, 14, 17, 19, 20]])

_SC_PARAMS = dict(
    compiler_params=pltpu.CompilerParams(use_tc_tiling_on_sc=False,
                                         needs_layout_passes=False),
)


@functools.cache
def _mesh():
    return plsc.VectorSubcoreMesh(core_axis_name="c", subcore_axis_name="s")


def _posgather_kernel(pos_hbm, s_hbm, r_hbm, ps_hbm, pr_hbm,
                      idxs, idxr, bufs, bufr, sem1, sem2):
    c = lax.axis_index("c")
    sid = lax.axis_index("s")
    wid = sid * 2 + c
    e_pad = s_hbm.shape[0]
    per_w = e_pad // 32
    nchunks = per_w // 128

    def issue(i, b):
        base = wid * per_w + i * 256 + b * 128
        pltpu.sync_copy(s_hbm.at[pl.ds(base, 128)], idxs.at[b])
        pltpu.sync_copy(r_hbm.at[pl.ds(base, 128)], idxr.at[b])
        sem = sem1 if b == 0 else sem2
        pltpu.async_copy(pos_hbm.at[idxs.at[b]], bufs.at[b], sem)
        pltpu.async_copy(pos_hbm.at[idxr.at[b]], bufr.at[b], sem)

    def drain(b):
        sem = sem1 if b == 0 else sem2
        pltpu.make_async_copy(pos_hbm.at[idxs.at[b]], bufs.at[b], sem).wait()
        pltpu.make_async_copy(pos_hbm.at[idxr.at[b]], bufr.at[b], sem).wait()

    issue(0, 0)
    issue(0, 1)

    def body(j, carry):
        for b in range(2):
            base = wid * per_w + j * 256 + b * 128
            drain(b)
            pltpu.sync_copy(bufs.at[b], ps_hbm.at[pl.ds(base, 128)])
            pltpu.sync_copy(bufr.at[b], pr_hbm.at[pl.ds(base, 128)])

            @pl.when(j + 1 < nchunks // 2)
            def _():
                issue(j + 1, b)
        return carry

    lax.fori_loop(0, nchunks // 2, body, 0)


def _posgather(pos_pad, s_pad, r_pad):
    e_pad = s_pad.shape[0]
    out = jax.ShapeDtypeStruct((e_pad, 16), jnp.float32)
    k = pl.kernel(
        _posgather_kernel,
        out_type=(out, out),
        mesh=_mesh(),
        scratch_types=[
            pltpu.VMEM((2, 128), jnp.int32),
            pltpu.VMEM((2, 128), jnp.int32),
            pltpu.VMEM((2, 128, 16), jnp.float32),
            pltpu.VMEM((2, 128, 16), jnp.float32),
            pltpu.SemaphoreType.DMA,
            pltpu.SemaphoreType.DMA,
        ],
        **_SC_PARAMS,
    )
    return k(pos_pad, s_pad, r_pad)


def _edgeproj_kernel(ps_ref, pr_ref, ea_ref, w_ref, b_ref, *out_refs):
    d = pr_ref[...] - ps_ref[...]
    ss = jnp.sum(d * d, axis=1, keepdims=True)
    ln = jnp.sqrt(ss)
    inv = 1.0 / (ln + 1e-9)
    u = d * inv
    lane = lax.broadcasted_iota(jnp.int32, d.shape, 1)
    x = jnp.where(lane < 3, u,
                  jnp.where(lane == 3, ln,
                            jnp.where(lane == 4, ea_ref[...], 0.0)))
    p = lax.dot_general(x, w_ref[...], (((1,), (0,)), ((), ())),
                        preferred_element_type=jnp.float32) + b_ref[...]
    pb = p.astype(jnp.bfloat16)
    for q, oref in enumerate(out_refs):
        oref[...] = pb[:, 32 * q:32 * q + 32]


def _edgeproj(ps, pr, ea_pad, w_pad, b_pad):
    e_pad = ps.shape[0]
    blk = 4096
    grid = e_pad // blk
    out = [jax.ShapeDtypeStruct((e_pad, 32), jnp.bfloat16) for _ in range(6)]
    return pl.pallas_call(
        _edgeproj_kernel,
        grid=(grid,),
        in_specs=[
            pl.BlockSpec((blk, 16), lambda i: (i, 0)),
            pl.BlockSpec((blk, 16), lambda i: (i, 0)),
            pl.BlockSpec((blk, 1), lambda i: (i, 0)),
            pl.BlockSpec((16, 192), lambda i: (0, 0)),
            pl.BlockSpec((1, 192), lambda i: (0, 0)),
        ],
        out_specs=[pl.BlockSpec((blk, 32), lambda i: (i, 0)) for _ in range(6)],
        out_shape=out,
    )(ps, pr, ea_pad, w_pad, b_pad)


def _embed_kernel(na_ref, w_ref, b_ref, s0, s1, r0, r1):
    t = (na_ref[...] * w_ref[...] + b_ref[...]).astype(jnp.bfloat16)
    s0[...] = t[:, 0:32]
    s1[...] = t[:, 32:64]
    r0[...] = t[:, 64:96]
    r1[...] = t[:, 96:128]


def _embed_tables(node_attrs, w_comb, b_comb):
    n = node_attrs.shape[0]
    out = [jax.ShapeDtypeStruct((n, 32), jnp.bfloat16) for _ in range(4)]
    return pl.pallas_call(
        _embed_kernel,
        grid=(n // _BN,),
        in_specs=[
            pl.BlockSpec((_BN, 1), lambda i: (i, 0)),
            pl.BlockSpec((1, 128), lambda i: (0, 0)),
            pl.BlockSpec((1, 128), lambda i: (0, 0)),
        ],
        out_specs=[pl.BlockSpec((_BN, 32), lambda i: (i, 0)) for _ in range(4)],
        out_shape=out,
    )(node_attrs, w_comb, b_comb)


def _nodeproj_kernel(xp_ref, y0_ref, y1_ref, w_ref, xo, s0, s1, r0, r1):
    x = xp_ref[...] + jnp.concatenate([y0_ref[...], y1_ref[...]], axis=1)
    xo[...] = x
    t = lax.dot_general(x, w_ref[...], (((1,), (0,)), ((), ())),
                        preferred_element_type=jnp.float32).astype(jnp.bfloat16)
    s0[...] = t[:, 0:32]
    s1[...] = t[:, 32:64]
    r0[...] = t[:, 64:96]
    r1[...] = t[:, 96:128]


def _node_tables(xprev, y0, y1, w_all):
    n = xprev.shape[0]
    out = ([jax.ShapeDtypeStruct((n, 32), jnp.float32)] +
           [jax.ShapeDtypeStruct((n, 32), jnp.bfloat16) for _ in range(4)])
    return pl.pallas_call(
        _nodeproj_kernel,
        grid=(n // _BN,),
        in_specs=[
            pl.BlockSpec((_BN, 32), lambda i: (i, 0)),
            pl.BlockSpec((_BN, 16), lambda i: (i, 0)),
            pl.BlockSpec((_BN, 16), lambda i: (i, 0)),
            pl.BlockSpec((32, 128), lambda i: (0, 0)),
        ],
        out_specs=[pl.BlockSpec((_BN, 32), lambda i: (i, 0))] +
                  [pl.BlockSpec((_BN, 32), lambda i: (i, 0)) for _ in range(4)],
        out_shape=out,
    )(xprev, y0, y1, w_all)


def _softplus_sc(gv):
    t = jnp.exp(gv)
    y = 1.0 + t
    bi = plsc.bitcast(y, jnp.int32)
    ex = (bi >> 23) - 127
    mb = plsc.bitcast((bi & 0x007FFFFF) | 0x3F800000, jnp.float32)
    big = mb > 1.4142135
    m2 = jnp.where(big, mb * 0.5, mb)
    ef = ex.astype(jnp.float32) + jnp.where(big, 1.0, 0.0)
    sf = (m2 - 1.0) / (m2 + 1.0)
    z = sf * sf
    lm = sf * (2.0 + z * (0.6666666667 + z * (0.4 + z * 0.2857142857)))
    ly = ef * 0.69314718056 + lm
    return jnp.where(gv > 15.0, gv, ly)


def _mp_kernel(s_hbm, r_hbm, sx0, sx1, rx0, rx1, e0_hbm, e1_hbm,
               y0_hbm, y1_hbm,
               idxs, idxr, bufS, bufR, bufE, bufMsg, zbuf, acc,
               sem0, sem1):
    c = lax.axis_index("c")
    sid = lax.axis_index("s")
    e_pad = s_hbm.shape[0]
    n_acc = acc.shape[0]
    per_tile = e_pad // 16
    nchunks = per_tile // 128
    rows_per_tile = n_acc // 16
    zchunk = rows_per_tile // 64
    sems = (sem0, sem1)

    def run(s_ref, r_ref, e_ref, y_ref):
        def zfill(i, carry):
            zbuf[i, :] = jnp.zeros((16,), jnp.float32)
            return carry
        lax.fori_loop(0, zchunk, zfill, 0)

        def zcopy(j, carry):
            pltpu.sync_copy(zbuf, acc.at[pl.ds(sid * rows_per_tile + j * zchunk, zchunk)])
            return carry
        lax.fori_loop(0, 64, zcopy, 0)
        plsc.subcore_barrier()

        def issue(i, b):
            base = sid * per_tile + i * 128
            pltpu.sync_copy(s_hbm.at[pl.ds(base, 128)], idxs.at[b])
            pltpu.sync_copy(r_hbm.at[pl.ds(base, 128)], idxr.at[b])
            pltpu.async_copy(s_ref.at[idxs.at[b]], bufS.at[b], sems[b])
            pltpu.async_copy(r_ref.at[idxr.at[b]], bufR.at[b], sems[b])
            pltpu.async_copy(e_ref.at[pl.ds(base, 128)], bufE.at[b], sems[b])

        def drain(b):
            pltpu.make_async_copy(s_ref.at[idxs.at[b]], bufS.at[b], sems[b]).wait()
            pltpu.make_async_copy(r_ref.at[idxr.at[b]], bufR.at[b], sems[b]).wait()
            pltpu.make_async_copy(e_ref.at[pl.ds(0, 128)], bufE.at[b], sems[b]).wait()

        issue(0, 0)
        issue(1, 1)

        def body(j, carry):
            for b in range(2):
                drain(b)

                def edge(e, carry2):
                    def vm(w):
                        v = plsc.bitcast(w << 16, jnp.float32)
                        m = plsc.bitcast(w & jnp.int32(-65536), jnp.float32)
                        return v, m

                    av, am = vm(plsc.bitcast(bufS[b, e, :], jnp.int32))
                    bv, bm = vm(plsc.bitcast(bufR[b, e, :], jnp.int32))
                    ev, em = vm(plsc.bitcast(bufE[b, e, :], jnp.int32))
                    gv = av + bv + ev
                    gm = am + bm + em
                    sg = 1.0 / (1.0 + jnp.exp(-gm))
                    bufMsg[e, :] = _softplus_sc(gv) * sg
                    return carry2

                lax.fori_loop(0, 128, edge, 0)
                pltpu.sync_copy(bufMsg, acc.at[idxr.at[b]], add=True)

                @pl.when(2 * j + 2 + b < nchunks)
                def _():
                    issue(2 * j + 2 + b, b)
            return carry

        lax.fori_loop(0, nchunks // 2, body, 0)
        plsc.subcore_barrier()
        pltpu.sync_copy(acc.at[pl.ds(sid * rows_per_tile, rows_per_tile)],
                        y_ref.at[pl.ds(sid * rows_per_tile, rows_per_tile)])

    @pl.when(c == 0)
    def _():
        run(sx0, rx0, e0_hbm, y0_hbm)

    @pl.when(c == 1)
    def _():
        run(sx1, rx1, e1_hbm, y1_hbm)


def _mp_layer(s_pad, r_pad, sx0, sx1, rx0, rx1, e0, e1, n_t):
    out = jax.ShapeDtypeStruct((n_t, 16), jnp.float32)
    k = pl.kernel(
        _mp_kernel,
        out_type=(out, out),
        mesh=_mesh(),
        scratch_types=[
            pltpu.VMEM((2, 128), jnp.int32),
            pltpu.VMEM((2, 128), jnp.int32),
            pltpu.VMEM((2, 128, 32), jnp.bfloat16),
            pltpu.VMEM((2, 128, 32), jnp.bfloat16),
            pltpu.VMEM((2, 128, 32), jnp.bfloat16),
            pltpu.VMEM((128, 16), jnp.float32),
            pltpu.VMEM((n_t // 1024, 16), jnp.float32),
            pltpu.VMEM_SHARED((n_t, 16), jnp.float32),
            pltpu.SemaphoreType.DMA,
            pltpu.SemaphoreType.DMA,
        ],
        **_SC_PARAMS,
    )
    return k(s_pad, r_pad, sx0, sx1, rx0, rx1, e0, e1)


def _head_kernel(nblocks, xp_ref, y0_ref, y1_ref, b_ref,
                 w1, b1, w2, b2, w3, b3, w4, b4, out_ref, sums, cnts):
    i = pl.program_id(0)

    @pl.when(i == 0)
    def _():
        sums[...] = jnp.zeros_like(sums)
        cnts[...] = jnp.zeros_like(cnts)

    x = xp_ref[...] + jnp.concatenate([y0_ref[...], y1_ref[...]], axis=1)
    g = lax.broadcasted_iota(jnp.int32, (x.shape[0], 256), 1)
    oh = (g == b_ref[...]).astype(jnp.float32)
    sums[...] += lax.dot_general(oh, x, (((0,), (0,)), ((), ())),
                                 preferred_element_type=jnp.float32)
    cnts[...] += lax.dot_general(oh, jnp.ones_like(x), (((0,), (0,)), ((), ())),
                                 preferred_element_type=jnp.float32)

    @pl.when(i == nblocks - 1)
    def _():
        gf = sums[...] / jnp.maximum(cnts[...], 1.0)

        def sp(v):
            return jnp.maximum(v, 0.0) + jnp.log1p(jnp.exp(-jnp.abs(v)))

        h = sp(jnp.dot(gf, w1[...], preferred_element_type=jnp.float32) + b1[...])
        h = sp(jnp.dot(h, w2[...], preferred_element_type=jnp.float32) + b2[...])
        h = sp(jnp.dot(h, w3[...], preferred_element_type=jnp.float32) + b3[...])
        out_ref[...] = jnp.dot(h, w4[...], preferred_element_type=jnp.float32) + b4[...]


def _head(xprev, y0, y1, batch2d, w1, b1, w2, b2, w3, b3, w4, b4):
    n = batch2d.shape[0]
    bn = 1000
    nblocks = n // bn
    return pl.pallas_call(
        functools.partial(_head_kernel, nblocks),
        grid=(nblocks,),
        in_specs=[
            pl.BlockSpec((bn, 32), lambda i: (i, 0)),
            pl.BlockSpec((bn, 16), lambda i: (i, 0)),
            pl.BlockSpec((bn, 16), lambda i: (i, 0)),
            pl.BlockSpec((bn, 1), lambda i: (i, 0)),
            pl.BlockSpec((32, 128), lambda i: (0, 0)),
            pl.BlockSpec((1, 128), lambda i: (0, 0)),
            pl.BlockSpec((128, 64), lambda i: (0, 0)),
            pl.BlockSpec((1, 64), lambda i: (0, 0)),
            pl.BlockSpec((64, 32), lambda i: (0, 0)),
            pl.BlockSpec((1, 32), lambda i: (0, 0)),
            pl.BlockSpec((32, 21), lambda i: (0, 0)),
            pl.BlockSpec((1, 21), lambda i: (0, 0)),
        ],
        out_specs=pl.BlockSpec((256, 21), lambda i: (0, 0)),
        out_shape=jax.ShapeDtypeStruct((256, 21), jnp.float32),
        scratch_shapes=[
            pltpu.VMEM((256, 32), jnp.float32),
            pltpu.VMEM((256, 32), jnp.float32),
        ],
    )(xprev, y0, y1, batch2d, w1, b1, w2, b2, w3, b3, w4, b4)


def kernel(node_attrs, positions, shifts, edge_attr, edge_index, batch_ids,
           W_node, b_node, W_edge, b_edge, Wv, bv, Wm, bm,
           W1, b1, W2, b2, W3, b3, W4, b4):
    n = node_attrs.shape[0]
    e = edge_index.shape[1]
    e_pad = ((e + 4095) // 4096) * 4096
    mp = Wv.shape[0]
    n_t = _NT

    s = edge_index[0].astype(jnp.int32)
    r = edge_index[1].astype(jnp.int32)
    pad = e_pad - e
    pad_idx = jnp.full((pad,), n, jnp.int32)
    s_pad = jnp.concatenate([s, pad_idx])
    r_pad = jnp.concatenate([r, pad_idx])
    pos_pad = jnp.pad(positions, ((0, n_t - n), (0, 13)))
    na_pad = jnp.pad(node_attrs, ((0, n_t - n), (0, 0)))
    ea_pad = jnp.pad(edge_attr, ((0, pad), (0, 0)))

    wp_cols = []
    bp_cols = []
    for l in range(mp):
        we_v = W_edge @ Wv[l][2 * _HID:3 * _HID]
        be_v = b_edge @ Wv[l][2 * _HID:3 * _HID] + bv[l]
        we_m = W_edge @ Wm[l][2 * _HID:3 * _HID]
        be_m = b_edge @ Wm[l][2 * _HID:3 * _HID] + bm[l]
        for h in range(2):
            wp_cols.append(jnp.stack([we_v[:, 16 * h:16 * h + 16],
                                      we_m[:, 16 * h:16 * h + 16]], axis=2).reshape(5, 32))
            bp_cols.append(jnp.stack([be_v[16 * h:16 * h + 16],
                                      be_m[16 * h:16 * h + 16]], axis=1).reshape(32))
    w_pe = jnp.concatenate(wp_cols, axis=1)
    w_pad = jnp.zeros((16, 192), jnp.float32).at[0:5, :].set(w_pe)
    b_pad = jnp.concatenate(bp_cols).reshape(1, 192)

    def table_w(l):
        ws_v, wr_v = Wv[l][0:_HID], Wv[l][_HID:2 * _HID]
        ws_m, wr_m = Wm[l][0:_HID], Wm[l][_HID:2 * _HID]
        cols = []
        for a, bcol in ((ws_v, ws_m), (wr_v, wr_m)):
            for h in range(2):
                cols.append(jnp.stack([a[:, 16 * h:16 * h + 16],
                                       bcol[:, 16 * h:16 * h + 16]], axis=2).reshape(_HID, 32))
        return jnp.concatenate(cols, axis=1)

    w_tab = [table_w(l) for l in range(mp)]
    w_comb = (W_node @ w_tab[0]).reshape(1, 128)
    b_comb = (b_node @ w_tab[0]).reshape(1, 128)

    ps, pr = _posgather(pos_pad, s_pad, r_pad)
    slabs = _edgeproj(ps, pr, ea_pad, w_pad, b_pad)

    sx0, sx1, rx0, rx1 = _embed_tables(na_pad, w_comb, b_comb)
    y0, y1 = _mp_layer(s_pad, r_pad, sx0, sx1, rx0, rx1,
                       slabs[0], slabs[1], n_t)
    xprev = jnp.zeros((n_t, _HID), jnp.float32)
    for l in range(1, mp):
        xprev, sx0, sx1, rx0, rx1 = _node_tables(xprev, y0, y1, w_tab[l])
        y0, y1 = _mp_layer(s_pad, r_pad, sx0, sx1, rx0, rx1,
                           slabs[2 * l], slabs[2 * l + 1], n_t)

    out = _head(xprev, y0, y1, batch_ids.astype(jnp.int32).reshape(n, 1),
                W1, b1.reshape(1, -1), W2, b2.reshape(1, -1),
                W3, b3.reshape(1, -1), W4, b4.reshape(1, -1))
    return out[:, _INDS]


# speedup vs baseline: 3.1175x; 1.0068x over previous
import functools

import jax
import jax.numpy as jnp
import numpy as np
from jax import lax
from jax.experimental import pallas as pl
from jax.experimental.pallas import tpu as pltpu
from jax.experimental.pallas import tpu_sc as plsc

_HID = 32
_HALF = 16
_NT = 100352
_BN = 1024
_INDS = np.array([[0, 1, 2, 3, 4, 5], [1, 6, 7, 8, 9, 10], [2, 7, 11, 12, 13, 14],
                  [3, 8, 12, 15, 16, 17], [4, 9, 13, 16, 18, 19], [5, 10, 14, 17, 19, 20]])

_SC_PARAMS = dict(
    compiler_params=pltpu.CompilerParams(use_tc_tiling_on_sc=False,
                                         needs_layout_passes=False),
)


@functools.cache
def _mesh():
    return plsc.VectorSubcoreMesh(core_axis_name="c", subcore_axis_name="s")


def _posgather_kernel(pos_hbm, s_hbm, r_hbm, ps_hbm, pr_hbm,
                      idxs, idxr, bufs, bufr, sem1, sem2):
    c = lax.axis_index("c")
    sid = lax.axis_index("s")
    wid = sid * 2 + c
    e_pad = s_hbm.shape[0]
    per_w = e_pad // 32
    nchunks = per_w // 128

    def issue(i, b):
        base = wid * per_w + i * 256 + b * 128
        pltpu.sync_copy(s_hbm.at[pl.ds(base, 128)], idxs.at[b])
        pltpu.sync_copy(r_hbm.at[pl.ds(base, 128)], idxr.at[b])
        sem = sem1 if b == 0 else sem2
        pltpu.async_copy(pos_hbm.at[idxs.at[b]], bufs.at[b], sem)
        pltpu.async_copy(pos_hbm.at[idxr.at[b]], bufr.at[b], sem)

    def drain(b):
        sem = sem1 if b == 0 else sem2
        pltpu.make_async_copy(pos_hbm.at[idxs.at[b]], bufs.at[b], sem).wait()
        pltpu.make_async_copy(pos_hbm.at[idxr.at[b]], bufr.at[b], sem).wait()

    issue(0, 0)
    issue(0, 1)

    def body(j, carry):
        for b in range(2):
            base = wid * per_w + j * 256 + b * 128
            drain(b)
            pltpu.sync_copy(bufs.at[b], ps_hbm.at[pl.ds(base, 128)])
            pltpu.sync_copy(bufr.at[b], pr_hbm.at[pl.ds(base, 128)])

            @pl.when(j + 1 < nchunks // 2)
            def _():
                issue(j + 1, b)
        return carry

    lax.fori_loop(0, nchunks // 2, body, 0)


def _posgather(pos_pad, s_pad, r_pad):
    e_pad = s_pad.shape[0]
    out = jax.ShapeDtypeStruct((e_pad, 8), jnp.float32)
    k = pl.kernel(
        _posgather_kernel,
        out_type=(out, out),
        mesh=_mesh(),
        scratch_types=[
            pltpu.VMEM((2, 128), jnp.int32),
            pltpu.VMEM((2, 128), jnp.int32),
            pltpu.VMEM((2, 128, 8), jnp.float32),
            pltpu.VMEM((2, 128, 8), jnp.float32),
            pltpu.SemaphoreType.DMA,
            pltpu.SemaphoreType.DMA,
        ],
        **_SC_PARAMS,
    )
    return k(pos_pad, s_pad, r_pad)


def _edgeproj_kernel(ps_ref, pr_ref, ea_ref, w_ref, b_ref, *out_refs):
    d = pr_ref[...] - ps_ref[...]
    ss = jnp.sum(d * d, axis=1, keepdims=True)
    ln = jnp.sqrt(ss)
    inv = 1.0 / (ln + 1e-9)
    u = d * inv
    x = jnp.concatenate([u, jnp.zeros_like(u)], axis=1)
    lane = lax.broadcasted_iota(jnp.int32, x.shape, 1)
    x = jnp.where(lane == 3, ln, x)
    x = jnp.where(lane == 4, ea_ref[...], x)
    p = lax.dot_general(x, w_ref[...], (((1,), (0,)), ((), ())),
                        preferred_element_type=jnp.float32) + b_ref[...]
    pb = p.astype(jnp.bfloat16)
    for q, oref in enumerate(out_refs):
        oref[...] = pb[:, 32 * q:32 * q + 32]


def _edgeproj(ps, pr, ea_pad, w_pad, b_pad):
    e_pad = ps.shape[0]
    blk = 8192
    grid = e_pad // blk
    out = [jax.ShapeDtypeStruct((e_pad, 32), jnp.bfloat16) for _ in range(6)]
    return pl.pallas_call(
        _edgeproj_kernel,
        grid=(grid,),
        in_specs=[
            pl.BlockSpec((blk, 8), lambda i: (i, 0)),
            pl.BlockSpec((blk, 8), lambda i: (i, 0)),
            pl.BlockSpec((blk, 1), lambda i: (i, 0)),
            pl.BlockSpec((16, 192), lambda i: (0, 0)),
            pl.BlockSpec((1, 192), lambda i: (0, 0)),
        ],
        out_specs=[pl.BlockSpec((blk, 32), lambda i: (i, 0)) for _ in range(6)],
        out_shape=out,
    )(ps, pr, ea_pad, w_pad, b_pad)


def _embed_kernel(na_ref, w_ref, b_ref, s0, s1, r0, r1):
    t = (na_ref[...] * w_ref[...] + b_ref[...]).astype(jnp.bfloat16)
    s0[...] = t[:, 0:32]
    s1[...] = t[:, 32:64]
    r0[...] = t[:, 64:96]
    r1[...] = t[:, 96:128]


def _embed_tables(node_attrs, w_comb, b_comb):
    n = node_attrs.shape[0]
    out = [jax.ShapeDtypeStruct((n, 32), jnp.bfloat16) for _ in range(4)]
    return pl.pallas_call(
        _embed_kernel,
        grid=(n // _BN,),
        in_specs=[
            pl.BlockSpec((_BN, 1), lambda i: (i, 0)),
            pl.BlockSpec((1, 128), lambda i: (0, 0)),
            pl.BlockSpec((1, 128), lambda i: (0, 0)),
        ],
        out_specs=[pl.BlockSpec((_BN, 32), lambda i: (i, 0)) for _ in range(4)],
        out_shape=out,
    )(node_attrs, w_comb, b_comb)


def _nodeproj_kernel(xp_ref, y0_ref, y1_ref, w_ref, xo, s0, s1, r0, r1):
    x = xp_ref[...] + jnp.concatenate([y0_ref[...], y1_ref[...]], axis=1)
    xo[...] = x
    t = lax.dot_general(x, w_ref[...], (((1,), (0,)), ((), ())),
                        preferred_element_type=jnp.float32).astype(jnp.bfloat16)
    s0[...] = t[:, 0:32]
    s1[...] = t[:, 32:64]
    r0[...] = t[:, 64:96]
    r1[...] = t[:, 96:128]


def _node_tables(xprev, y0, y1, w_all):
    n = xprev.shape[0]
    out = ([jax.ShapeDtypeStruct((n, 32), jnp.float32)] +
           [jax.ShapeDtypeStruct((n, 32), jnp.bfloat16) for _ in range(4)])
    return pl.pallas_call(
        _nodeproj_kernel,
        grid=(n // _BN,),
        in_specs=[
            pl.BlockSpec((_BN, 32), lambda i: (i, 0)),
            pl.BlockSpec((_BN, 16), lambda i: (i, 0)),
            pl.BlockSpec((_BN, 16), lambda i: (i, 0)),
            pl.BlockSpec((32, 128), lambda i: (0, 0)),
        ],
        out_specs=[pl.BlockSpec((_BN, 32), lambda i: (i, 0))] +
                  [pl.BlockSpec((_BN, 32), lambda i: (i, 0)) for _ in range(4)],
        out_shape=out,
    )(xprev, y0, y1, w_all)


def _softplus_sc(gv):
    t = jnp.exp(gv)
    y = 1.0 + t
    bi = plsc.bitcast(y, jnp.int32)
    ex = (bi >> 23) - 127
    mb = plsc.bitcast((bi & 0x007FFFFF) | 0x3F800000, jnp.float32)
    big = mb > 1.4142135
    m2 = jnp.where(big, mb * 0.5, mb)
    ef = ex.astype(jnp.float32) + jnp.where(big, 1.0, 0.0)
    sf = (m2 - 1.0) / (m2 + 1.0)
    z = sf * sf
    lm = sf * (2.0 + z * (0.6666666667 + z * (0.4 + z * 0.2857142857)))
    ly = ef * 0.69314718056 + lm
    return jnp.where(gv > 15.0, gv, ly)


def _mp_kernel(s_hbm, r_hbm, sx0, sx1, rx0, rx1, e0_hbm, e1_hbm,
               y0_hbm, y1_hbm,
               idxs, idxr, bufS, bufR, bufE, bufMsg, zbuf, acc,
               sem0, sem1):
    c = lax.axis_index("c")
    sid = lax.axis_index("s")
    e_pad = s_hbm.shape[0]
    n_acc = acc.shape[0]
    per_tile = e_pad // 16
    nchunks = per_tile // 128
    rows_per_tile = n_acc // 16
    zchunk = rows_per_tile // 64
    sems = (sem0, sem1)

    def run(s_ref, r_ref, e_ref, y_ref):
        def zfill(i, carry):
            zbuf[i, :] = jnp.zeros((16,), jnp.float32)
            return carry
        lax.fori_loop(0, zchunk, zfill, 0)

        def zcopy(j, carry):
            pltpu.sync_copy(zbuf, acc.at[pl.ds(sid * rows_per_tile + j * zchunk, zchunk)])
            return carry
        lax.fori_loop(0, 64, zcopy, 0)
        plsc.subcore_barrier()

        def issue(i, b):
            base = sid * per_tile + i * 128
            pltpu.sync_copy(s_hbm.at[pl.ds(base, 128)], idxs.at[b])
            pltpu.sync_copy(r_hbm.at[pl.ds(base, 128)], idxr.at[b])
            pltpu.async_copy(s_ref.at[idxs.at[b]], bufS.at[b], sems[b])
            pltpu.async_copy(r_ref.at[idxr.at[b]], bufR.at[b], sems[b])
            pltpu.async_copy(e_ref.at[pl.ds(base, 128)], bufE.at[b], sems[b])

        def drain(b):
            pltpu.make_async_copy(s_ref.at[idxs.at[b]], bufS.at[b], sems[b]).wait()
            pltpu.make_async_copy(r_ref.at[idxr.at[b]], bufR.at[b], sems[b]).wait()
            pltpu.make_async_copy(e_ref.at[pl.ds(0, 128)], bufE.at[b], sems[b]).wait()

        issue(0, 0)
        issue(1, 1)

        def body(j, carry):
            for b in range(2):
                drain(b)

                def edge(e, carry2):
                    def vm(w):
                        v = plsc.bitcast(w << 16, jnp.float32)
                        m = plsc.bitcast(w & jnp.int32(-65536), jnp.float32)
                        return v, m

                    av, am = vm(plsc.bitcast(bufS[b, e, :], jnp.int32))
                    bv, bm = vm(plsc.bitcast(bufR[b, e, :], jnp.int32))
                    ev, em = vm(plsc.bitcast(bufE[b, e, :], jnp.int32))
                    gv = av + bv + ev
                    gm = am + bm + em
                    sg = 1.0 / (1.0 + jnp.exp(-gm))
                    bufMsg[e, :] = _softplus_sc(gv) * sg
                    return carry2

                lax.fori_loop(0, 128, edge, 0)
                pltpu.sync_copy(bufMsg, acc.at[idxr.at[b]], add=True)

                @pl.when(2 * j + 2 + b < nchunks)
                def _():
                    issue(2 * j + 2 + b, b)
            return carry

        lax.fori_loop(0, nchunks // 2, body, 0)
        plsc.subcore_barrier()
        pltpu.sync_copy(acc.at[pl.ds(sid * rows_per_tile, rows_per_tile)],
                        y_ref.at[pl.ds(sid * rows_per_tile, rows_per_tile)])

    @pl.when(c == 0)
    def _():
        run(sx0, rx0, e0_hbm, y0_hbm)

    @pl.when(c == 1)
    def _():
        run(sx1, rx1, e1_hbm, y1_hbm)


def _mp_layer(s_pad, r_pad, sx0, sx1, rx0, rx1, e0, e1, n_t):
    out = jax.ShapeDtypeStruct((n_t, 16), jnp.float32)
    k = pl.kernel(
        _mp_kernel,
        out_type=(out, out),
        mesh=_mesh(),
        scratch_types=[
            pltpu.VMEM((2, 128), jnp.int32),
            pltpu.VMEM((2, 128), jnp.int32),
            pltpu.VMEM((2, 128, 32), jnp.bfloat16),
            pltpu.VMEM((2, 128, 32), jnp.bfloat16),
            pltpu.VMEM((2, 128, 32), jnp.bfloat16),
            pltpu.VMEM((128, 16), jnp.float32),
            pltpu.VMEM((n_t // 1024, 16), jnp.float32),
            pltpu.VMEM_SHARED((n_t, 16), jnp.float32),
            pltpu.SemaphoreType.DMA,
            pltpu.SemaphoreType.DMA,
        ],
        **_SC_PARAMS,
    )
    return k(s_pad, r_pad, sx0, sx1, rx0, rx1, e0, e1)


def _head_kernel(nblocks, xp_ref, y0_ref, y1_ref, b_ref,
                 w1, b1, w2, b2, w3, b3, w4, b4, out_ref, sums, cnts):
    i = pl.program_id(0)

    @pl.when(i == 0)
    def _():
        sums[...] = jnp.zeros_like(sums)
        cnts[...] = jnp.zeros_like(cnts)

    x = xp_ref[...] + jnp.concatenate([y0_ref[...], y1_ref[...]], axis=1)
    g = lax.broadcasted_iota(jnp.int32, (x.shape[0], 256), 1)
    oh = (g == b_ref[...]).astype(jnp.float32)
    sums[...] += lax.dot_general(oh, x, (((0,), (0,)), ((), ())),
                                 preferred_element_type=jnp.float32)
    cnts[...] += lax.dot_general(oh, jnp.ones_like(x), (((0,), (0,)), ((), ())),
                                 preferred_element_type=jnp.float32)

    @pl.when(i == nblocks - 1)
    def _():
        gf = sums[...] / jnp.maximum(cnts[...], 1.0)

        def sp(v):
            return jnp.maximum(v, 0.0) + jnp.log1p(jnp.exp(-jnp.abs(v)))

        h = sp(jnp.dot(gf, w1[...], preferred_element_type=jnp.float32) + b1[...])
        h = sp(jnp.dot(h, w2[...], preferred_element_type=jnp.float32) + b2[...])
        h = sp(jnp.dot(h, w3[...], preferred_element_type=jnp.float32) + b3[...])
        out_ref[...] = jnp.dot(h, w4[...], preferred_element_type=jnp.float32) + b4[...]


def _head(xprev, y0, y1, batch2d, w1, b1, w2, b2, w3, b3, w4, b4):
    n = batch2d.shape[0]
    bn = 1000
    nblocks = n // bn
    return pl.pallas_call(
        functools.partial(_head_kernel, nblocks),
        grid=(nblocks,),
        in_specs=[
            pl.BlockSpec((bn, 32), lambda i: (i, 0)),
            pl.BlockSpec((bn, 16), lambda i: (i, 0)),
            pl.BlockSpec((bn, 16), lambda i: (i, 0)),
            pl.BlockSpec((bn, 1), lambda i: (i, 0)),
            pl.BlockSpec((32, 128), lambda i: (0, 0)),
            pl.BlockSpec((1, 128), lambda i: (0, 0)),
            pl.BlockSpec((128, 64), lambda i: (0, 0)),
            pl.BlockSpec((1, 64), lambda i: (0, 0)),
            pl.BlockSpec((64, 32), lambda i: (0, 0)),
            pl.BlockSpec((1, 32), lambda i: (0, 0)),
            pl.BlockSpec((32, 21), lambda i: (0, 0)),
            pl.BlockSpec((1, 21), lambda i: (0, 0)),
        ],
        out_specs=pl.BlockSpec((256, 21), lambda i: (0, 0)),
        out_shape=jax.ShapeDtypeStruct((256, 21), jnp.float32),
        scratch_shapes=[
            pltpu.VMEM((256, 32), jnp.float32),
            pltpu.VMEM((256, 32), jnp.float32),
        ],
    )(xprev, y0, y1, batch2d, w1, b1, w2, b2, w3, b3, w4, b4)


def kernel(node_attrs, positions, shifts, edge_attr, edge_index, batch_ids,
           W_node, b_node, W_edge, b_edge, Wv, bv, Wm, bm,
           W1, b1, W2, b2, W3, b3, W4, b4):
    n = node_attrs.shape[0]
    e = edge_index.shape[1]
    e_pad = ((e + 8191) // 8192) * 8192
    mp = Wv.shape[0]
    n_t = _NT

    s = edge_index[0].astype(jnp.int32)
    r = edge_index[1].astype(jnp.int32)
    pad = e_pad - e
    pad_idx = jnp.full((pad,), n, jnp.int32)
    s_pad = jnp.concatenate([s, pad_idx])
    r_pad = jnp.concatenate([r, pad_idx])
    pos_pad = jnp.pad(positions, ((0, n_t - n), (0, 5)))
    na_pad = jnp.pad(node_attrs, ((0, n_t - n), (0, 0)))
    ea_pad = jnp.pad(edge_attr, ((0, pad), (0, 0)))

    wp_cols = []
    bp_cols = []
    for l in range(mp):
        we_v = W_edge @ Wv[l][2 * _HID:3 * _HID]
        be_v = b_edge @ Wv[l][2 * _HID:3 * _HID] + bv[l]
        we_m = W_edge @ Wm[l][2 * _HID:3 * _HID]
        be_m = b_edge @ Wm[l][2 * _HID:3 * _HID] + bm[l]
        for h in range(2):
            wp_cols.append(jnp.stack([we_v[:, 16 * h:16 * h + 16],
                                      we_m[:, 16 * h:16 * h + 16]], axis=2).reshape(5, 32))
            bp_cols.append(jnp.stack([be_v[16 * h:16 * h + 16],
                                      be_m[16 * h:16 * h + 16]], axis=1).reshape(32))
    w_pe = jnp.concatenate(wp_cols, axis=1)
    w_pad = jnp.zeros((16, 192), jnp.float32).at[0:5, :].set(w_pe)
    b_pad = jnp.concatenate(bp_cols).reshape(1, 192)

    def table_w(l):
        ws_v, wr_v = Wv[l][0:_HID], Wv[l][_HID:2 * _HID]
        ws_m, wr_m = Wm[l][0:_HID], Wm[l][_HID:2 * _HID]
        cols = []
        for a, bcol in ((ws_v, ws_m), (wr_v, wr_m)):
            for h in range(2):
                cols.append(jnp.stack([a[:, 16 * h:16 * h + 16],
                                       bcol[:, 16 * h:16 * h + 16]], axis=2).reshape(_HID, 32))
        return jnp.concatenate(cols, axis=1)

    w_tab = [table_w(l) for l in range(mp)]
    w_comb = (W_node @ w_tab[0]).reshape(1, 128)
    b_comb = (b_node @ w_tab[0]).reshape(1, 128)

    ps, pr = _posgather(pos_pad, s_pad, r_pad)
    slabs = _edgeproj(ps, pr, ea_pad, w_pad, b_pad)

    sx0, sx1, rx0, rx1 = _embed_tables(na_pad, w_comb, b_comb)
    y0, y1 = _mp_layer(s_pad, r_pad, sx0, sx1, rx0, rx1,
                       slabs[0], slabs[1], n_t)
    xprev = jnp.zeros((n_t, _HID), jnp.float32)
    for l in range(1, mp):
        xprev, sx0, sx1, rx0, rx1 = _node_tables(xprev, y0, y1, w_tab[l])
        y0, y1 = _mp_layer(s_pad, r_pad, sx0, sx1, rx0, rx1,
                           slabs[2 * l], slabs[2 * l + 1], n_t)

    out = _head(xprev, y0, y1, batch_ids.astype(jnp.int32).reshape(n, 1),
                W1, b1.reshape(1, -1), W2, b2.reshape(1, -1),
                W3, b3.reshape(1, -1), W4, b4.reshape(1, -1))
    return out[:, _INDS]
